# baseline (speedup 1.0000x reference)
import functools

import jax
import jax.numpy as jnp
from jax import lax
from jax.experimental import pallas as pl
from jax.experimental.pallas import tpu as pltpu
from jax.experimental.pallas import tpu_sc as plsc

B = 16384
D = 64
L = 16
_EPS = 1e-12

NUM_FILES = 1000000
NUM_TESTS = 100000

_info = plsc.get_sparse_core_info()
NC = _info.num_cores
NS = _info.num_subcores
NW = NC * NS
BPW = B // NW
CR = 128
NCHUNK = BPW // CR


def _rsqrt_newton(x):
    i = lax.bitcast_convert_type(x, jnp.int32)
    i = jnp.int32(0x5F3759DF) - lax.shift_right_arithmetic(i, 1)
    y = lax.bitcast_convert_type(i, jnp.float32)
    half_x = x * 0.5
    for _ in range(3):
        y = y * (1.5 - half_x * y * y)
    return y


def _make_sc_kernel():
    mesh = plsc.VectorSubcoreMesh(core_axis_name="c", subcore_axis_name="s")

    @functools.partial(
        pl.kernel,
        mesh=mesh,
        out_type=jax.ShapeDtypeStruct((B,), jnp.float32),
        compiler_params=pltpu.CompilerParams(needs_layout_passes=False),
        scratch_types=[
            pltpu.VMEM((BPW,), jnp.int32),
            pltpu.VMEM((BPW,), jnp.int32),
            pltpu.VMEM((BPW,), jnp.int32),
            pltpu.VMEM((BPW,), jnp.int32),
            pltpu.VMEM((2, CR, 2 * D), jnp.float32),
            pltpu.VMEM((2, CR, 2 * D), jnp.float32),
            pltpu.VMEM((BPW,), jnp.float32),
            pltpu.SemaphoreType.DMA,
            pltpu.SemaphoreType.DMA,
        ],
    )
    def sc_kernel(fidx_hbm, tidx_hbm, ftab_hbm, ttab_hbm, out_hbm,
                  fidx_v, tidx_v, frow_v, trow_v,
                  fbuf_v, tbuf_v, out_v, sem0, sem1):
        wid = lax.axis_index("s") * NC + lax.axis_index("c")
        base = wid * BPW

        pltpu.sync_copy(fidx_hbm.at[pl.ds(base, BPW)], fidx_v)
        pltpu.sync_copy(tidx_hbm.at[pl.ds(base, BPW)], tidx_v)

        def rowidx_body(g, _):
            sl = pl.ds(g * L, L)
            frow_v[sl] = lax.shift_right_logical(fidx_v[sl], 1)
            trow_v[sl] = lax.shift_right_logical(tidx_v[sl], 1)
            return 0

        lax.fori_loop(0, BPW // L, rowidx_body, 0)

        sems = (sem0, sem1)

        def fire(c, buf):
            csl = pl.ds(c * CR, CR)
            pltpu.async_copy(
                ftab_hbm.at[frow_v.at[csl]], fbuf_v.at[buf], sems[buf])
            pltpu.async_copy(
                ttab_hbm.at[trow_v.at[csl]], tbuf_v.at[buf], sems[buf])

        def drain(buf):
            pltpu.make_async_copy(
                ftab_hbm.at[pl.ds(0, CR)], fbuf_v.at[buf], sems[buf]).wait()
            pltpu.make_async_copy(
                ttab_hbm.at[pl.ds(0, CR)], tbuf_v.at[buf], sems[buf]).wait()

        lane = lax.iota(jnp.int32, L)

        fire(0, 0)

        def chunk_body(c, buf):
            if c + 1 < NCHUNK:
                fire(c + 1, (c + 1) % 2)
            drain(buf)
            fbuf = fbuf_v.at[buf]
            tbuf = tbuf_v.at[buf]
            for g in range(CR // L):
                gsl = pl.ds(c * CR + g * L, L)
                fvec = fidx_v[gsl]
                tvec = tidx_v[gsl]
                acc_dot = jnp.zeros((L,), jnp.float32)
                acc_nf = jnp.zeros((L,), jnp.float32)
                acc_nt = jnp.zeros((L,), jnp.float32)
                for j in range(L):
                    i = g * L + j
                    fo = (fvec[j] & 1) * D
                    to = (tvec[j] & 1) * D
                    f = [fbuf[i, pl.ds(fo + k * L, L)] for k in range(D // L)]
                    t = [tbuf[i, pl.ds(to + k * L, L)] for k in range(D // L)]
                    p_dot = f[0] * t[0]
                    p_nf = f[0] * f[0]
                    p_nt = t[0] * t[0]
                    for k in range(1, D // L):
                        p_dot = p_dot + f[k] * t[k]
                        p_nf = p_nf + f[k] * f[k]
                        p_nt = p_nt + t[k] * t[k]
                    m = lane == j
                    acc_dot = jnp.where(m, jnp.sum(p_dot), acc_dot)
                    acc_nf = jnp.where(m, jnp.sum(p_nf), acc_nf)
                    acc_nt = jnp.where(m, jnp.sum(p_nt), acc_nt)
                denom = jnp.maximum(acc_nf, _EPS) * jnp.maximum(acc_nt, _EPS)
                out_v[gsl] = acc_dot * _rsqrt_newton(denom)

        for c in range(NCHUNK):
            chunk_body(c, c % 2)

        pltpu.sync_copy(out_v, out_hbm.at[pl.ds(base, BPW)])

    return sc_kernel


_sc_kernel = _make_sc_kernel()


@jax.jit
def kernel(file, test, file_table, test_table):
    ftab2 = file_table.reshape(NUM_FILES // 2, 2 * D)
    ttab2 = test_table.reshape(NUM_TESTS // 2, 2 * D)
    out = _sc_kernel(file.reshape(B), test.reshape(B), ftab2, ttab2)
    return out.reshape(B, 1)


# speedup vs baseline: 2.2320x; 2.2320x over previous
import functools

import jax
import jax.numpy as jnp
from jax import lax
from jax.experimental import pallas as pl
from jax.experimental.pallas import tpu as pltpu
from jax.experimental.pallas import tpu_sc as plsc

B = 16384
D = 64
L = 16
_EPS = 1e-12

NUM_FILES = 1000000
NUM_TESTS = 100000

_info = plsc.get_sparse_core_info()
NC = _info.num_cores
NS = _info.num_subcores
NW = NC * NS
BPW = B // NW
CR = 16
NCHUNK = BPW // CR


def _rsqrt_newton(x):
    i = lax.bitcast_convert_type(x, jnp.int32)
    i = jnp.int32(0x5F3759DF) - lax.shift_right_arithmetic(i, 1)
    y = lax.bitcast_convert_type(i, jnp.float32)
    half_x = x * 0.5
    for _ in range(3):
        y = y * (1.5 - half_x * y * y)
    return y


def _make_sc_kernel():
    mesh = plsc.VectorSubcoreMesh(core_axis_name="c", subcore_axis_name="s")

    @functools.partial(
        pl.kernel,
        mesh=mesh,
        out_type=jax.ShapeDtypeStruct((B,), jnp.float32),
        compiler_params=pltpu.CompilerParams(needs_layout_passes=False),
        scratch_types=[
            pltpu.VMEM((BPW,), jnp.int32),
            pltpu.VMEM((BPW,), jnp.int32),
            pltpu.VMEM((2, CR, 8, D), jnp.float32),
            pltpu.VMEM((2, CR, 8, D), jnp.float32),
            pltpu.VMEM((BPW,), jnp.float32),
            pltpu.SemaphoreType.DMA,
        ],
    )
    def sc_kernel(fidx_hbm, tidx_hbm, ftab_hbm, ttab_hbm, out_hbm,
                  fidx_v, tidx_v, fbuf_v, tbuf_v, out_v, sem):
        wid = lax.axis_index("s") * NC + lax.axis_index("c")
        base = wid * BPW

        pltpu.sync_copy(fidx_hbm.at[pl.ds(base, BPW)], fidx_v)
        pltpu.sync_copy(tidx_hbm.at[pl.ds(base, BPW)], tidx_v)

        lane = lax.iota(jnp.int32, L)

        def fire(c, buf):
            gsl = pl.ds(c * CR, CR)
            fvec = lax.shift_right_logical(fidx_v[gsl], 3)
            tvec = lax.shift_right_logical(tidx_v[gsl], 3)
            for j in range(CR):
                pltpu.async_copy(
                    ftab_hbm.at[fvec[j]], fbuf_v.at[buf, j], sem)
                pltpu.async_copy(
                    ttab_hbm.at[tvec[j]], tbuf_v.at[buf, j], sem)

        def drain(buf):
            pltpu.make_async_copy(
                ftab_hbm.at[pl.ds(0, CR)], fbuf_v.at[buf], sem).wait()
            pltpu.make_async_copy(
                ttab_hbm.at[pl.ds(0, CR)], tbuf_v.at[buf], sem).wait()

        fire(jnp.int32(0), jnp.int32(0))

        def chunk_body(c, _):
            buf = lax.rem(c, 2)

            @pl.when(c + 1 < NCHUNK)
            def _():
                fire(c + 1, lax.rem(c + 1, 2))

            drain(buf)
            gsl = pl.ds(c * CR, CR)
            fvec = fidx_v[gsl] & 7
            tvec = tidx_v[gsl] & 7
            acc_dot = jnp.zeros((L,), jnp.float32)
            acc_nf = jnp.zeros((L,), jnp.float32)
            acc_nt = jnp.zeros((L,), jnp.float32)
            for j in range(CR):
                fs = fvec[j]
                ts = tvec[j]
                f = [fbuf_v[buf, j, fs, pl.ds(k * L, L)]
                     for k in range(D // L)]
                t = [tbuf_v[buf, j, ts, pl.ds(k * L, L)]
                     for k in range(D // L)]
                p_dot = f[0] * t[0]
                p_nf = f[0] * f[0]
                p_nt = t[0] * t[0]
                for k in range(1, D // L):
                    p_dot = p_dot + f[k] * t[k]
                    p_nf = p_nf + f[k] * f[k]
                    p_nt = p_nt + t[k] * t[k]
                m = lane == j
                acc_dot = jnp.where(m, jnp.sum(p_dot), acc_dot)
                acc_nf = jnp.where(m, jnp.sum(p_nf), acc_nf)
                acc_nt = jnp.where(m, jnp.sum(p_nt), acc_nt)
            denom = jnp.maximum(acc_nf, _EPS) * jnp.maximum(acc_nt, _EPS)
            out_v[gsl] = acc_dot * _rsqrt_newton(denom)
            return 0

        lax.fori_loop(0, NCHUNK, chunk_body, 0)

        pltpu.sync_copy(out_v, out_hbm.at[pl.ds(base, BPW)])

    return sc_kernel


_sc_kernel = _make_sc_kernel()


@jax.jit
def kernel(file, test, file_table, test_table):
    ftab3 = file_table.reshape(NUM_FILES // 8, 8, D)
    ttab3 = test_table.reshape(NUM_TESTS // 8, 8, D)
    out = _sc_kernel(file.reshape(B), test.reshape(B), ftab3, ttab3)
    return out.reshape(B, 1)


# speedup vs baseline: 2.3561x; 1.0556x over previous
import functools

import jax
import jax.numpy as jnp
from jax import lax
from jax.experimental import pallas as pl
from jax.experimental.pallas import tpu as pltpu
from jax.experimental.pallas import tpu_sc as plsc

B = 16384
D = 64
L = 16
_EPS = 1e-12

NUM_FILES = 1000000
NUM_TESTS = 100000

_info = plsc.get_sparse_core_info()
NC = _info.num_cores
NS = _info.num_subcores
NW = NC * NS
BPW = B // NW
NG = BPW // L
SUB = 2
NSUB = L // SUB
TH = 8
NH = BPW // TH


def _rsqrt_newton(x):
    i = lax.bitcast_convert_type(x, jnp.int32)
    i = jnp.int32(0x5F3759DF) - lax.shift_right_arithmetic(i, 1)
    y = lax.bitcast_convert_type(i, jnp.float32)
    half_x = x * 0.5
    for _ in range(3):
        y = y * (1.5 - half_x * y * y)
    return y


def _make_sc_kernel():
    mesh = plsc.VectorSubcoreMesh(core_axis_name="c", subcore_axis_name="s")

    @functools.partial(
        pl.kernel,
        mesh=mesh,
        out_type=jax.ShapeDtypeStruct((B,), jnp.float32),
        compiler_params=pltpu.CompilerParams(needs_layout_passes=False),
        scratch_types=[
            pltpu.VMEM((BPW,), jnp.int32),
            pltpu.VMEM((BPW + L,), jnp.int32),
            pltpu.VMEM((2, SUB, 8, 8, 128), jnp.float32),
            pltpu.VMEM((2, TH, 8, D), jnp.float32),
            pltpu.VMEM((BPW // 2, 2 * D), jnp.float32),
            pltpu.VMEM((BPW,), jnp.float32),
            pltpu.SemaphoreType.DMA,
            pltpu.SemaphoreType.DMA,
        ],
    )
    def sc_kernel(fidx_hbm, tidx_hbm, ftabd_hbm, ttab3_hbm, out_hbm,
                  fidx_v, tidx_v, fblk_v, tbuf_v, frows_v,
                  out_v, semf, semt):
        wid = lax.axis_index("s") * NC + lax.axis_index("c")
        base = wid * BPW

        pltpu.sync_copy(fidx_hbm.at[pl.ds(base, BPW)], fidx_v)
        pltpu.sync_copy(tidx_hbm.at[pl.ds(base, BPW)],
                        tidx_v.at[pl.ds(0, BPW)])

        lane16 = lax.iota(jnp.int32, L)

        ivecs = [(jnp.int32(2 * k) + lax.shift_right_logical(lane16, 3))
                 for k in range(D // L)]
        svec = lane16 & 7

        def f_fire(jblk, buf, rr):
            off = pl.multiple_of(jblk * 128, 128)
            pltpu.async_copy(
                ftabd_hbm.at[slice(None), slice(None), pl.ds(off, 128)],
                fblk_v.at[buf, rr], semf)

        def f_drain(buf, rr):
            pltpu.make_async_copy(
                ftabd_hbm.at[slice(None), slice(None), pl.ds(0, 128)],
                fblk_v.at[buf, rr], semf).wait()

        def f_body(g, _):
            fvec = fidx_v[pl.ds(g * L, L)]
            jvec = lax.shift_right_logical(fvec, 7)
            lvec = fvec & 127

            @pl.when(g == 0)
            def _():
                for rr in range(SUB):
                    f_fire(jvec[rr], jnp.int32(0), rr)

            for q in range(NSUB):
                buf = lax.rem(g * NSUB + q, 2)
                nbuf = lax.rem(g * NSUB + q + 1, 2)
                if q + 1 < NSUB:
                    for rr in range(SUB):
                        f_fire(jvec[(q + 1) * SUB + rr], nbuf, rr)
                else:
                    @pl.when(g + 1 < NG)
                    def _():
                        nvec = lax.shift_right_logical(
                            fidx_v[pl.ds((g + 1) * L, L)], 7)
                        for rr in range(SUB):
                            f_fire(nvec[rr], nbuf, rr)

                for rr in range(SUB):
                    f_drain(buf, rr)
                for rr in range(SUB):
                    j = q * SUB + rr
                    lanev = jnp.full((L,), lvec[j], jnp.int32)
                    for k in range(D // L):
                        v = plsc.load_gather(
                            fblk_v,
                            [jnp.full((L,), buf, jnp.int32),
                             jnp.full((L,), rr, jnp.int32),
                             ivecs[k], svec, lanev])
                        frows_v[g * 8 + q, pl.ds(rr * D + k * L, L)] = v
            return 0

        lax.fori_loop(0, NG, f_body, 0)

        def t_fire(h, buf):
            tvec = lax.shift_right_logical(tidx_v[pl.ds(h * TH, L)], 3)
            for j in range(TH):
                pltpu.async_copy(
                    ttab3_hbm.at[tvec[j]], tbuf_v.at[buf, j], semt)

        def t_drain(buf):
            pltpu.make_async_copy(
                ttab3_hbm.at[pl.ds(0, TH)], tbuf_v.at[buf], semt).wait()

        t_fire(jnp.int32(0), jnp.int32(0))

        def h_body(h, acc):
            acc_dot, acc_nf, acc_nt = acc
            buf = lax.rem(h, 2)

            @pl.when(h + 1 < NH)
            def _():
                t_fire(h + 1, lax.rem(h + 1, 2))

            t_drain(buf)
            half = lax.rem(h, 2)
            is_even = half == 0
            acc_dot = jnp.where(is_even, 0.0, acc_dot)
            acc_nf = jnp.where(is_even, 0.0, acc_nf)
            acc_nt = jnp.where(is_even, 0.0, acc_nt)
            tvec = tidx_v[pl.ds(h * TH, L)] & 7
            for j in range(TH):
                ts = tvec[j]
                frow = h * (TH // 2) + (j >> 1)
                fcol = (j & 1) * D
                f = [frows_v[frow, pl.ds(fcol + k * L, L)]
                     for k in range(D // L)]
                t = [tbuf_v[buf, j, ts, pl.ds(k * L, L)]
                     for k in range(D // L)]
                p_dot = f[0] * t[0]
                p_nf = f[0] * f[0]
                p_nt = t[0] * t[0]
                for k in range(1, D // L):
                    p_dot = p_dot + f[k] * t[k]
                    p_nf = p_nf + f[k] * f[k]
                    p_nt = p_nt + t[k] * t[k]
                m = lane16 == (half * TH + j)
                acc_dot = jnp.where(m, jnp.sum(p_dot), acc_dot)
                acc_nf = jnp.where(m, jnp.sum(p_nf), acc_nf)
                acc_nt = jnp.where(m, jnp.sum(p_nt), acc_nt)

            @pl.when(half == 1)
            def _():
                denom = (jnp.maximum(acc_nf, _EPS)
                         * jnp.maximum(acc_nt, _EPS))
                gbase = lax.shift_right_logical(h, 1) * L
                out_v[pl.ds(gbase, L)] = acc_dot * _rsqrt_newton(denom)

            return (acc_dot, acc_nf, acc_nt)

        zeros = jnp.zeros((L,), jnp.float32)
        lax.fori_loop(0, NH, h_body, (zeros, zeros, zeros))

        pltpu.sync_copy(out_v, out_hbm.at[pl.ds(base, BPW)])

    return sc_kernel


_sc_kernel = _make_sc_kernel()


@jax.jit
def kernel(file, test, file_table, test_table):
    ftabd = file_table.T.reshape(8, 8, NUM_FILES)
    ttab3 = test_table.reshape(NUM_TESTS // 8, 8, D)
    out = _sc_kernel(file.reshape(B), test.reshape(B), ftabd, ttab3)
    return out.reshape(B, 1)
